# baseline (speedup 1.0000x reference)
import jax
import jax.numpy as jnp
from jax import lax
from jax.experimental import pallas as pl
from jax.experimental.pallas import tpu as pltpu
from jax.experimental.pallas import tpu_sc as plsc

_F = 17
_H = 61
_W = 81
_N = _H * _W
_AH = 481
_AHP = 488
_AW = 641
_R = 4
_SROWS = 80
_BR = 96
_BC = 672
_FSTRIDE = _AHP * _BC
_SSTRIDE = _SROWS * _BC
_NSTRIP = 6
_NTASK = _F * _NSTRIP
_MAGIC = 12582912.0


def _sc_body(x_hbm, xout_hbm, acc_hbm, chan_v, wl_v, buf, copy_sem):
    wid = lax.axis_index("s") * 2 + lax.axis_index("c")
    lanes = lax.iota(jnp.int32, 16)
    zero16f = jnp.zeros((16,), jnp.float32)
    zero16i = jnp.zeros((16,), jnp.int32)

    def prefix_incl(hit):
        x = hit.astype(jnp.int32)
        for d in (1, 2, 4, 8):
            g = x[jnp.maximum(lanes - d, 0)]
            x = x + jnp.where(lanes >= d, g, 0)
        return x

    @pl.when(wid == 31)
    def _start_copy():
        pltpu.make_async_copy(x_hbm, xout_hbm, copy_sem).start()

    for t in range(4):
        i = wid + 32 * t
        live = i < _NTASK

        @pl.when(live)
        def _task(i=i):
            s = lax.shift_right_logical(i * 3856, 16)
            f = i - 17 * s
            y_lo = jnp.where(s == 0, -8, 80 * s)
            y_hi = jnp.where(s == 5, 488, 80 * s + 79)
            y_lo_v = jnp.broadcast_to(y_lo, (16,))
            y_hi_v = jnp.broadcast_to(y_hi, (16,))
            sofs_v = jnp.broadcast_to(80 * s, (16,))

            pltpu.sync_copy(x_hbm.at[f + 1, pl.ds(0, 3)],
                            chan_v.at[pl.ds(0, 3)])
            pltpu.sync_copy(x_hbm.at[f + 1, 4], chan_v.at[3])

            def zero_grp(g, c):
                buf[pl.ds(16 * g, 16)] = zero16f
                return c
            lax.fori_loop(0, _BR * _BC // 16, zero_grp, 0)

            def scan_row(r, cnt):
                for c0 in (0, 16, 32, 48, 64):
                    colv = c0 + lanes
                    v = chan_v[0, r, pl.ds(c0, 16)]
                    cyy = chan_v[2, r, pl.ds(c0, 16)] * 8.0
                    scc = chan_v[3, r, pl.ds(c0, 16)] * 8.0
                    valid = (v > 0.1) & (scc >= 0.0)
                    ry = jnp.clip(cyy, -100.0, 584.0)
                    ry = (ry + _MAGIC) - _MAGIC
                    cy0 = jnp.clip(ry, -4.0, 484.0).astype(jnp.int32)
                    hit = valid & (cy0 + _R >= y_lo_v) & (cy0 - _R <= y_hi_v)
                    pos = cnt + prefix_incl(hit) - 1
                    plsc.store_scatter(wl_v, [pos], jnp.broadcast_to(r * _W, (16,)) + colv, mask=hit)
                    cnt = cnt + plsc.all_reduce_population_count(hit)
                return cnt
            cnt = lax.fori_loop(0, _H, scan_row, zero16i)
            col80 = 80 + zero16i
            for r0 in (0, 16, 32, 48):
                rows = r0 + lanes
                rows_c = jnp.minimum(rows, _H - 1)
                v = plsc.load_gather(chan_v, [zero16i, rows_c, col80])
                cyy = plsc.load_gather(chan_v, [2 + zero16i, rows_c, col80]) * 8.0
                scc = plsc.load_gather(chan_v, [3 + zero16i, rows_c, col80]) * 8.0
                valid = (rows < _H) & (v > 0.1) & (scc >= 0.0)
                ry = jnp.clip(cyy, -100.0, 584.0)
                ry = (ry + _MAGIC) - _MAGIC
                cy0 = jnp.clip(ry, -4.0, 484.0).astype(jnp.int32)
                hit = valid & (cy0 + _R >= y_lo_v) & (cy0 - _R <= y_hi_v)
                pos = cnt + prefix_incl(hit) - 1
                plsc.store_scatter(wl_v, [pos], rows * _W + 80, mask=hit)
                cnt = cnt + plsc.all_reduce_population_count(hit)

            def cond(j):
                return jnp.any(j < cnt)

            def point(j):
                pidx = plsc.load_gather(wl_v, [j])
                pr = lax.shift_right_logical(pidx * 12946, 20)
                pc = pidx - pr * _W
                v = plsc.load_gather(chan_v, [zero16i, pr, pc])
                cxx = plsc.load_gather(chan_v, [1 + zero16i, pr, pc]) * 8.0
                cyy = plsc.load_gather(chan_v, [2 + zero16i, pr, pc]) * 8.0
                scc = plsc.load_gather(chan_v, [3 + zero16i, pr, pc]) * 8.0
                sig = jnp.maximum(1.0, 0.5 * scc)
                sig2 = sig * sig
                val = v * 0.0625
                rx = jnp.clip(cxx, -100.0, 744.0)
                rx = (rx + _MAGIC) - _MAGIC
                cx0 = jnp.clip(rx, -4.0, 644.0).astype(jnp.int32)
                ry = jnp.clip(cyy, -100.0, 584.0)
                ry = (ry + _MAGIC) - _MAGIC
                cy0 = jnp.clip(ry, -4.0, 484.0).astype(jnp.int32)
                def cell_group(u, carry):
                    k = u * 16 + lanes
                    dyq = lax.shift_right_logical(k * 57, 9)
                    dy = dyq - 4
                    dx = k - 9 * dyq - 4
                    xx = cx0 + dx
                    yy = cy0 + dy
                    xxc = jnp.clip(xx, 0, _AW - 1)
                    yyc = jnp.clip(yy, 0, _AH - 1)
                    fdx = xxc.astype(jnp.float32) - cxx
                    fdy = yyc.astype(jnp.float32) - cyy
                    dx2 = fdx * fdx
                    dy2 = fdy * fdy
                    d2 = dx2 + dy2
                    nearest = (dx2 < 0.25) & (dy2 < 0.25)
                    w = jnp.where(nearest, val,
                                  val * jnp.exp((-0.5 * d2) / sig2))
                    m = (d2 <= sig2) & (k < 81) & (yy >= y_lo_v) & (yy <= y_hi_v)
                    row = jnp.where(yy < 0, yy + _BR, yy - sofs_v)
                    row = jnp.clip(row, 0, _BR - 1)
                    col = jnp.where(xx < 0, xx + 657, xx)
                    plsc.addupdate_scatter(buf, [row * _BC + col], w, mask=m)
                    return carry
                lax.fori_loop(0, 6, cell_group, 0)
                return j + 1
            lax.while_loop(cond, point, zero16i)

            any_pts = jnp.any(cnt > 0)

            @pl.when(any_pts & (s == 0))
            def _fold_top():
                for k in range(_BC // 16):
                    acc_v = buf[pl.ds(16 * k, 16)]
                    for h in range(88, 96):
                        acc_v = acc_v + buf[pl.ds(h * _BC + 16 * k, 16)]
                    buf[pl.ds(16 * k, 16)] = acc_v

            @pl.when(any_pts & (s == 5))
            def _fold_bottom():
                for k in range(_BC // 16):
                    acc_v = buf[pl.ds(80 * _BC + 16 * k, 16)]
                    for h in range(81, 89):
                        acc_v = acc_v + buf[pl.ds(h * _BC + 16 * k, 16)]
                    buf[pl.ds(80 * _BC + 16 * k, 16)] = acc_v

            @pl.when(any_pts)
            def _fold_x_and_clamp():
                def fold_rows(rr, c):
                    rows = (rr * 16 + lanes) * _BC
                    left = plsc.load_gather(buf, [rows])
                    for cc in range(649, 657):
                        left = left + plsc.load_gather(buf, [rows + cc])
                    plsc.store_scatter(buf, [rows], left)
                    right = plsc.load_gather(buf, [rows + 640])
                    for cc in range(641, 649):
                        right = right + plsc.load_gather(buf, [rows + cc])
                    plsc.store_scatter(buf, [rows + 640], right)
                    return c
                lax.fori_loop(0, 6, fold_rows, 0)

                def clamp_grp(g, c):
                    q = buf[pl.ds(16 * g, 16)]
                    buf[pl.ds(16 * g, 16)] = jnp.minimum(q, 1.0)
                    return c
                lax.fori_loop(0, 88 * _BC // 16, clamp_grp, 0)

            base = f * _FSTRIDE + s * _SSTRIDE

            @pl.when(s < 5)
            def _out_main():
                pltpu.sync_copy(buf.at[pl.ds(0, _SSTRIDE)],
                                acc_hbm.at[pl.ds(base, _SSTRIDE)])

            @pl.when(s == 5)
            def _out_last():
                pltpu.sync_copy(buf.at[pl.ds(0, 88 * _BC)],
                                acc_hbm.at[pl.ds(base, 88 * _BC)])

    @pl.when(wid == 31)
    def _drain_copy():
        pltpu.make_async_copy(x_hbm, xout_hbm, copy_sem).wait()


_sc_call = pl.kernel(
    _sc_body,
    out_type=(
        jax.ShapeDtypeStruct((18, 5, _H, _W), jnp.float32),
        jax.ShapeDtypeStruct((_F * _FSTRIDE,), jnp.float32),
    ),
    mesh=plsc.VectorSubcoreMesh(core_axis_name="c", subcore_axis_name="s",
                                num_cores=2, num_subcores=16),
    compiler_params=pltpu.CompilerParams(needs_layout_passes=False, use_tc_tiling_on_sc=False),
    scratch_types=[
        pltpu.VMEM((4, _H, _W), jnp.float32),
        pltpu.VMEM((_N + 3,), jnp.int32),
        pltpu.VMEM((_BR * _BC,), jnp.float32),
        pltpu.SemaphoreType.DMA,
    ],
)


def kernel(x):
    out, _acc = _sc_call(x)
    return out
